# baseline (device time: 14417 ns/iter reference)
import os

import jax
import jax.numpy as jnp
from jax import lax
from jax.experimental import pallas as pl
from jax.experimental.pallas import tpu as pltpu

B = 8
H = 8
D = 64
BS = 16
NB = 64
NPAGES = 128
NPAGES_LOCAL = 64
HD = H * D
BH = B * H
ROWS = NPAGES_LOCAL * BS
SCALE = D ** -0.5
NEG = -1e30

_NO_COMM = bool(int(os.environ.get("SCB_NO_COMM", "0")))


def _body(kv_ref, qt_ref, lmt_ref, out_ref,
          kvf, rbuf, rrem, send_sems, recv_sems):
    my_x = lax.axis_index("x")
    my_y = lax.axis_index("y")
    my_z = lax.axis_index("z")
    nbr = (1 - my_x, my_y, my_z)

    if not _NO_COMM:
        barrier_sem = pltpu.get_barrier_semaphore()
        pl.semaphore_signal(barrier_sem, inc=1, device_id=nbr,
                            device_id_type=pl.DeviceIdType.MESH)
        pl.semaphore_wait(barrier_sem, 1)

    kvf[:, :] = kv_ref[:, :].astype(jnp.bfloat16)

    ecol = (lax.broadcasted_iota(jnp.int32, (B, BH), 1) // H
            == lax.broadcasted_iota(jnp.int32, (B, BH), 0)
            ).astype(jnp.float32)
    hm = (lax.broadcasted_iota(jnp.int32, (HD, BH), 0) // D
          == lax.broadcasted_iota(jnp.int32, (HD, BH), 1) % H
          ).astype(jnp.float32)
    r16 = (lax.broadcasted_iota(jnp.int32, (ROWS, NPAGES_LOCAL), 0) // BS
           == lax.broadcasted_iota(jnp.int32, (ROWS, NPAGES_LOCAL), 1)
           ).astype(jnp.float32)

    qmall = (lax.dot_general(
        qt_ref[:, :], ecol, (((1,), (0,)), ((), ())),
        preferred_element_type=jnp.float32,
    ) * hm).astype(jnp.bfloat16)
    lm8 = lax.dot_general(
        r16, lmt_ref[:, :], (((1,), (0,)), ((), ())),
        preferred_element_type=jnp.float32,
    )
    lm = lax.dot_general(
        lm8, ecol, (((1,), (0,)), ((), ())),
        preferred_element_type=jnp.float32,
    )

    GC = BH // 2
    GR = GC + 8
    rdmas = []
    for g in range(2):
        cols = slice(g * GC, (g + 1) * GC)
        lmg = lm[:, cols]
        s = lax.dot_general(
            kvf[:, 0:HD], qmall[:, cols], (((1,), (0,)), ((), ())),
            preferred_element_type=jnp.float32,
        ) * SCALE + lmg
        m = jnp.max(s, axis=0, keepdims=True)
        p_ = jnp.where(lmg > 0.5 * NEG, jnp.exp(s - m), 0.0)
        d = jnp.sum(p_, axis=0, keepdims=True)
        r = lax.dot_general(
            p_.astype(jnp.bfloat16), kvf[:, HD:2 * HD],
            (((0,), (0,)), ((), ())),
            preferred_element_type=jnp.float32,
        )
        base = g * GR
        rbuf[base:base + GC, :] = r
        rbuf[base + GC:base + GC + 1, 0:GC] = m
        rbuf[base + GC + 1:base + GC + 2, 0:GC] = d

        if not _NO_COMM:
            rr = pltpu.make_async_remote_copy(
                src_ref=rbuf.at[pl.ds(base, GR), :],
                dst_ref=rrem.at[pl.ds(base, GR), :],
                send_sem=send_sems.at[g], recv_sem=recv_sems.at[g],
                device_id=nbr, device_id_type=pl.DeviceIdType.MESH,
            )
            rr.start()
            rdmas.append(rr)

    hsel = (lax.broadcasted_iota(jnp.int32, (H, HD), 1) // D
            == lax.broadcasted_iota(jnp.int32, (H, HD), 0)
            ).astype(jnp.float32)
    for g in range(2):
        if not _NO_COMM:
            rdmas[g].wait()
        base = g * GR
        m1 = rbuf[base + GC:base + GC + 1, 0:GC]
        d1 = rbuf[base + GC + 1:base + GC + 2, 0:GC]
        m2 = rrem[base + GC:base + GC + 1, 0:GC]
        d2 = rrem[base + GC + 1:base + GC + 2, 0:GC]
        mm = jnp.maximum(m1, m2)
        e1 = jnp.exp(m1 - mm)
        e2 = jnp.exp(m2 - mm)
        den = d1 * e1 + d2 * e2
        e1c = jnp.transpose(e1)
        e2c = jnp.transpose(e2)
        denc = jnp.transpose(den)
        merged = (rbuf[base:base + GC, :] * e1c
                  + rrem[base:base + GC, :] * e2c) / denc
        for ii in range(GC // H):
            i = g * (GC // H) + ii
            mi = merged[ii * H:(ii + 1) * H, :]
            out_ref[i:i + 1, :] = jnp.sum(mi * hsel, axis=0,
                                          keepdims=True)


def kernel(Q, K, V, bt, lens):
    kv2 = jnp.concatenate(
        [K.reshape(ROWS, HD), V.reshape(ROWS, HD)], axis=1
    )
    q2 = Q.reshape(B, HD)

    my_x = lax.axis_index("x")

    jmask = jnp.arange(NB, dtype=jnp.int32)[None, :] < lens[:, None]
    onehot = (bt[:, :, None] ==
              jnp.arange(NPAGES, dtype=jnp.int32)[None, None, :])
    cnt = jnp.sum(jnp.where(jmask[:, :, None], onehot, False)
                  .astype(jnp.float32), axis=1)
    cnt_my = lax.dynamic_slice(cnt, (0, my_x * NPAGES_LOCAL),
                               (B, NPAGES_LOCAL))
    logm = jnp.where(cnt_my > 0, jnp.log(cnt_my), NEG)
    logmt = logm.T
    qt = q2.T

    out2 = pl.pallas_call(
        _body,
        out_shape=jax.ShapeDtypeStruct((B, HD), jnp.float32),
        in_specs=[
            pl.BlockSpec(memory_space=pltpu.VMEM),
            pl.BlockSpec(memory_space=pltpu.VMEM),
            pl.BlockSpec(memory_space=pltpu.VMEM),
        ],
        out_specs=pl.BlockSpec(memory_space=pltpu.VMEM),
        scratch_shapes=[
            pltpu.VMEM((ROWS, 2 * HD), jnp.bfloat16),
            pltpu.VMEM((BH + 16, HD), jnp.float32),
            pltpu.VMEM((BH + 16, HD), jnp.float32),
            pltpu.SemaphoreType.DMA((2,)),
            pltpu.SemaphoreType.DMA((2,)),
        ],
        compiler_params=pltpu.CompilerParams(
            collective_id=None if _NO_COMM else 0
        ),
    )(kv2, qt, logmt)
    return out2.reshape(B, 1, H, D)


# device time: 13562 ns/iter; 1.0630x vs baseline; 1.0630x over previous
import os

import jax
import jax.numpy as jnp
from jax import lax
from jax.experimental import pallas as pl
from jax.experimental.pallas import tpu as pltpu

B = 8
H = 8
D = 64
BS = 16
NB = 64
NPAGES = 128
NPAGES_LOCAL = 64
HD = H * D
BH = B * H
ROWS = NPAGES_LOCAL * BS
SCALE = D ** -0.5
NEG = -1e30

_NO_COMM = bool(int(os.environ.get("SCB_NO_COMM", "0")))


def _body(kv_ref, qt_ref, lmt_ref, out_ref,
          kvf, rbuf, rrem, send_sems, recv_sems):
    my_x = lax.axis_index("x")
    my_y = lax.axis_index("y")
    my_z = lax.axis_index("z")
    nbr = (1 - my_x, my_y, my_z)

    if not _NO_COMM:
        barrier_sem = pltpu.get_barrier_semaphore()
        pl.semaphore_signal(barrier_sem, inc=1, device_id=nbr,
                            device_id_type=pl.DeviceIdType.MESH)
        pl.semaphore_wait(barrier_sem, 1)

    kvf[:, :] = kv_ref[:, :].astype(jnp.bfloat16)

    ecol = (lax.broadcasted_iota(jnp.int32, (B, BH), 1) // H
            == lax.broadcasted_iota(jnp.int32, (B, BH), 0)
            ).astype(jnp.float32)
    hm = (lax.broadcasted_iota(jnp.int32, (HD, BH), 0) // D
          == lax.broadcasted_iota(jnp.int32, (HD, BH), 1) % H
          ).astype(jnp.float32)
    r16 = (lax.broadcasted_iota(jnp.int32, (ROWS, NPAGES_LOCAL), 0) // BS
           == lax.broadcasted_iota(jnp.int32, (ROWS, NPAGES_LOCAL), 1)
           ).astype(jnp.float32)

    qmall = (lax.dot_general(
        qt_ref[:, :], ecol, (((1,), (0,)), ((), ())),
        preferred_element_type=jnp.float32,
    ) * hm).astype(jnp.bfloat16)
    lm8 = lax.dot_general(
        r16, lmt_ref[:, :], (((1,), (0,)), ((), ())),
        preferred_element_type=jnp.float32,
    )
    lm = lax.dot_general(
        lm8, ecol, (((1,), (0,)), ((), ())),
        preferred_element_type=jnp.float32,
    )

    s = lax.dot_general(
        kvf[:, 0:HD], qmall, (((1,), (0,)), ((), ())),
        preferred_element_type=jnp.float32,
    ) * SCALE + lm
    m = jnp.max(s, axis=0, keepdims=True)
    p_ = jnp.where(lm > 0.5 * NEG, jnp.exp(s - m), 0.0)
    d = jnp.sum(p_, axis=0, keepdims=True)
    r = lax.dot_general(
        p_.astype(jnp.bfloat16), kvf[:, HD:2 * HD],
        (((0,), (0,)), ((), ())),
        preferred_element_type=jnp.float32,
    )
    rbuf[0:BH, :] = r
    rbuf[BH:BH + 1, 0:BH] = m
    rbuf[BH + 1:BH + 2, 0:BH] = d

    if not _NO_COMM:
        rr = pltpu.make_async_remote_copy(
            src_ref=rbuf, dst_ref=rrem,
            send_sem=send_sems.at[0], recv_sem=recv_sems.at[0],
            device_id=nbr, device_id_type=pl.DeviceIdType.MESH,
        )
        rr.start()
        rr.wait()

    m1 = rbuf[BH:BH + 1, 0:BH]
    d1 = rbuf[BH + 1:BH + 2, 0:BH]
    m2 = rrem[BH:BH + 1, 0:BH]
    d2 = rrem[BH + 1:BH + 2, 0:BH]
    mm = jnp.maximum(m1, m2)
    e1 = jnp.exp(m1 - mm)
    e2 = jnp.exp(m2 - mm)
    den = d1 * e1 + d2 * e2
    e1c = jnp.transpose(e1)
    e2c = jnp.transpose(e2)
    denc = jnp.transpose(den)
    merged = (rbuf[0:BH, :] * e1c + rrem[0:BH, :] * e2c) / denc
    hsel = (lax.broadcasted_iota(jnp.int32, (H, HD), 1) // D
            == lax.broadcasted_iota(jnp.int32, (H, HD), 0)
            ).astype(jnp.float32)
    for i in range(B):
        mi = merged[i * H:(i + 1) * H, :]
        out_ref[i:i + 1, :] = jnp.sum(mi * hsel, axis=0, keepdims=True)


def kernel(Q, K, V, bt, lens):
    kv2 = jnp.concatenate(
        [K.reshape(ROWS, HD), V.reshape(ROWS, HD)], axis=1
    )
    q2 = Q.reshape(B, HD)

    my_x = lax.axis_index("x")

    jmask = jnp.arange(NB, dtype=jnp.int32)[None, :] < lens[:, None]
    onehot = (bt[:, :, None] ==
              jnp.arange(NPAGES, dtype=jnp.int32)[None, None, :])
    cnt = jnp.sum(jnp.where(jmask[:, :, None], onehot, False)
                  .astype(jnp.float32), axis=1)
    cnt_my = lax.dynamic_slice(cnt, (0, my_x * NPAGES_LOCAL),
                               (B, NPAGES_LOCAL))
    logm = jnp.where(cnt_my > 0, jnp.log(cnt_my), NEG)
    logmt = logm.T
    qt = q2.T

    out2 = pl.pallas_call(
        _body,
        out_shape=jax.ShapeDtypeStruct((B, HD), jnp.float32),
        in_specs=[
            pl.BlockSpec(memory_space=pltpu.VMEM),
            pl.BlockSpec(memory_space=pltpu.VMEM),
            pl.BlockSpec(memory_space=pltpu.VMEM),
        ],
        out_specs=pl.BlockSpec(memory_space=pltpu.VMEM),
        scratch_shapes=[
            pltpu.VMEM((ROWS, 2 * HD), jnp.bfloat16),
            pltpu.VMEM((BH + 2, HD), jnp.float32),
            pltpu.VMEM((BH + 2, HD), jnp.float32),
            pltpu.SemaphoreType.DMA((1,)),
            pltpu.SemaphoreType.DMA((1,)),
        ],
        compiler_params=pltpu.CompilerParams(
            collective_id=None if _NO_COMM else 0
        ),
    )(kv2, qt, logmt)
    return out2.reshape(B, 1, H, D)
